# baseline (speedup 1.0000x reference)
import functools

import jax
import jax.numpy as jnp
from jax import lax
from jax.experimental import pallas as pl
from jax.experimental.pallas import tpu as pltpu
from jax.experimental.pallas import tpu_sc as plsc

_N = 8192
_K = 256
_C = 8192
_COMMITMENT_COST = 0.25

_W = 4096
_BI = 512
_BJ = 1024
_NI = _N // _BI
_NJ = _C // _BJ
_JPW = _W // _BJ


def _bf16_round(v):
    return v.astype(jnp.bfloat16).astype(jnp.float32)


def _argmin_tile(x_ref, e_ref, x2_ref, e2_ref, idx_ref, val_ref,
                 wv, wi, sv, si, fv):
    j = pl.program_id(1)
    m = lax.dot_general(x_ref[...], e_ref[...], (((1,), (1,)), ((), ())),
                        preferred_element_type=jnp.float32)
    d = (x2_ref[...] + e2_ref[...]) - 2.0 * m
    lmin = jnp.min(d, axis=1)
    col = lax.broadcasted_iota(jnp.int32, d.shape, 1)
    larg = jnp.min(jnp.where(d == lmin[:, None], col, jnp.int32(2**30)),
                   axis=1)
    glob = larg + j * _BJ

    @pl.when(j % _JPW == 0)
    def _start_window():
        wv[0, :] = lmin
        wi[0, :] = glob

    @pl.when(j % _JPW != 0)
    def _merge_window():
        better = lmin < wv[0, :]
        wv[0, :] = jnp.where(better, lmin, wv[0, :])
        wi[0, :] = jnp.where(better, glob, wi[0, :])

    @pl.when(j == _JPW - 1)
    def _chain_init():
        sv[0, :] = _bf16_round(wv[0, :])
        si[0, :] = wi[0, :]
        fv[0, :] = wv[0, :]

    @pl.when(j == _NJ - 1)
    def _chain_fold_emit():
        v, w = wv[0, :], wi[0, :]
        take = (v < sv[0, :]) | ((v == sv[0, :]) & (w < si[0, :]))
        idx_ref[0, 0, :] = jnp.where(take, w, si[0, :])
        val_ref[0, 0, :] = jnp.where(take, v, fv[0, :])


def _argmin_call(flat, emb, x2, e2, interpret=False):
    return pl.pallas_call(
        _argmin_tile,
        grid=(_NI, _NJ),
        in_specs=[
            pl.BlockSpec((_BI, _K), lambda i, j: (i, 0)),
            pl.BlockSpec((_BJ, _K), lambda i, j: (j, 0)),
            pl.BlockSpec((_BI, 1), lambda i, j: (i, 0)),
            pl.BlockSpec((1, _BJ), lambda i, j: (0, j)),
        ],
        out_specs=[
            pl.BlockSpec((1, 1, _BI), lambda i, j: (i, 0, 0)),
            pl.BlockSpec((1, 1, _BI), lambda i, j: (i, 0, 0)),
        ],
        out_shape=[
            jax.ShapeDtypeStruct((_NI, 1, _BI), jnp.int32),
            jax.ShapeDtypeStruct((_NI, 1, _BI), jnp.float32),
        ],
        scratch_shapes=[
            pltpu.VMEM((1, _BI), jnp.float32),
            pltpu.VMEM((1, _BI), jnp.int32),
            pltpu.VMEM((1, _BI), jnp.float32),
            pltpu.VMEM((1, _BI), jnp.int32),
            pltpu.VMEM((1, _BI), jnp.float32),
        ],
        compiler_params=pltpu.CompilerParams(
            dimension_semantics=("parallel", "arbitrary")),
        interpret=interpret,
    )(flat, emb, x2, e2)


def _gather_rows(emb, idx):
    info = plsc.get_sparse_core_info()
    nc, ns = info.num_cores, info.num_subcores
    nw = nc * ns
    b_per_w = _N // nw
    mesh = plsc.VectorSubcoreMesh(core_axis_name="c", subcore_axis_name="s")

    @functools.partial(
        pl.kernel, mesh=mesh,
        out_type=jax.ShapeDtypeStruct((_N, _K), jnp.float32),
        scratch_types=[
            pltpu.VMEM((b_per_w,), jnp.int32),
            pltpu.VMEM((b_per_w, _K), jnp.float32),
            pltpu.SemaphoreType.DMA,
        ],
    )
    def gather_k(emb_hbm, idx_hbm, out_hbm, idx_v, rows_v, sem):
        wid = lax.axis_index("s") * nc + lax.axis_index("c")
        base = wid * b_per_w
        pltpu.sync_copy(idx_hbm.at[pl.ds(base, b_per_w)], idx_v)
        pltpu.async_copy(emb_hbm.at[idx_v], rows_v, sem).wait()
        pltpu.sync_copy(rows_v, out_hbm.at[pl.ds(base, b_per_w)])

    return gather_k(emb, idx)


def kernel(x, emb):
    flat = x.reshape(_N, _K)
    x2 = jnp.sum(flat ** 2, axis=1, keepdims=True)
    e2 = jnp.sum(emb ** 2, axis=1).reshape(1, _C)
    idx3, val3 = _argmin_call(flat, emb, x2, e2)
    idx = idx3.reshape(_N)
    minval = val3.reshape(_N)

    quantized = _gather_rows(emb, idx).reshape(x.shape)

    mse = jnp.sum(minval) / (_N * _K)
    loss = mse + _COMMITMENT_COST * mse
    quantized_st = x + (quantized - x)
    return quantized_st, loss, idx.reshape(x.shape[0], x.shape[1], x.shape[2])


# speedup vs baseline: 1.6449x; 1.6449x over previous
import functools

import jax
import jax.numpy as jnp
from jax import lax
from jax.experimental import pallas as pl
from jax.experimental.pallas import tpu as pltpu
from jax.experimental.pallas import tpu_sc as plsc

_N = 8192
_K = 256
_C = 8192
_COMMITMENT_COST = 0.25

_W = 4096
_BI = 512
_NI = _N // _BI
_BIG = 2 ** 30


def _bf16_round(v):
    return v.astype(jnp.bfloat16).astype(jnp.float32)


def _lane_tree_min(v, width):
    parts = [lax.slice_in_dim(v, k * 128, (k + 1) * 128, axis=1)
             for k in range(width // 128)]
    while len(parts) > 1:
        parts = ([jnp.minimum(a, b) for a, b in zip(parts[::2], parts[1::2])]
                 + ([parts[-1]] if len(parts) % 2 else []))
    return parts[0]


def _window_argmin(d, base):
    gm = _lane_tree_min(d, _W)
    wv = jnp.min(gm, axis=1)
    col = lax.broadcasted_iota(jnp.int32, d.shape, 1) + base
    ti = jnp.where(d == wv[:, None], col, _BIG)
    wi = jnp.min(_lane_tree_min(ti, _W), axis=1)
    return wv, wi


def _argmin_tile(x_ref, e_ref, x2_ref, e2_ref, idx_ref, val_ref):
    sv = si = fv = None
    for w in range(2):
        m2 = lax.dot_general(x_ref[...], e_ref[pl.ds(w * _W, _W), :],
                             (((1,), (1,)), ((), ())),
                             preferred_element_type=jnp.float32)
        t = x2_ref[...] + e2_ref[:, pl.ds(w * _W, _W)]
        d = t - m2
        wv, wi = _window_argmin(d, w * _W)
        if w == 0:
            sv, si, fv = _bf16_round(wv), wi, wv
        else:
            take = (wv < sv) | ((wv == sv) & (wi < si))
            idx_ref[0, 0, :] = jnp.where(take, wi, si)
            val_ref[0, 0, :] = jnp.where(take, wv, fv)


def _argmin_call(flat, emb2x, x2, e2, interpret=False):
    return pl.pallas_call(
        _argmin_tile,
        grid=(_NI,),
        in_specs=[
            pl.BlockSpec((_BI, _K), lambda i: (i, 0)),
            pl.BlockSpec((_C, _K), lambda i: (0, 0)),
            pl.BlockSpec((_BI, 1), lambda i: (i, 0)),
            pl.BlockSpec((1, _C), lambda i: (0, 0)),
        ],
        out_specs=[
            pl.BlockSpec((1, 1, _BI), lambda i: (i, 0, 0)),
            pl.BlockSpec((1, 1, _BI), lambda i: (i, 0, 0)),
        ],
        out_shape=[
            jax.ShapeDtypeStruct((_NI, 1, _BI), jnp.int32),
            jax.ShapeDtypeStruct((_NI, 1, _BI), jnp.float32),
        ],
        compiler_params=pltpu.CompilerParams(
            dimension_semantics=("arbitrary",)),
        interpret=interpret,
    )(flat, emb2x, x2, e2)


def _gather_rows(emb, idx):
    info = plsc.get_sparse_core_info()
    nc, ns = info.num_cores, info.num_subcores
    nw = nc * ns
    b_per_w = _N // nw
    mesh = plsc.VectorSubcoreMesh(core_axis_name="c", subcore_axis_name="s")

    @functools.partial(
        pl.kernel, mesh=mesh,
        out_type=jax.ShapeDtypeStruct((_N, _K), jnp.float32),
        scratch_types=[
            pltpu.VMEM((b_per_w,), jnp.int32),
            pltpu.VMEM((b_per_w, _K), jnp.float32),
            pltpu.SemaphoreType.DMA,
        ],
    )
    def gather_k(emb_hbm, idx_hbm, out_hbm, idx_v, rows_v, sem):
        wid = lax.axis_index("s") * nc + lax.axis_index("c")
        base = wid * b_per_w
        pltpu.sync_copy(idx_hbm.at[pl.ds(base, b_per_w)], idx_v)
        pltpu.async_copy(emb_hbm.at[idx_v], rows_v, sem).wait()
        pltpu.sync_copy(rows_v, out_hbm.at[pl.ds(base, b_per_w)])

    return gather_k(emb, idx)


def kernel(x, emb):
    flat = x.reshape(_N, _K)
    x2 = jnp.sum(flat ** 2, axis=1, keepdims=True)
    e2 = jnp.sum(emb ** 2, axis=1).reshape(1, _C)
    emb2x = emb + emb
    idx3, val3 = _argmin_call(flat, emb2x, x2, e2)
    idx = idx3.reshape(_N)
    minval = val3.reshape(_N)

    quantized = _gather_rows(emb, idx).reshape(x.shape)

    mse = jnp.sum(minval) / (_N * _K)
    loss = mse + _COMMITMENT_COST * mse
    quantized_st = x + (quantized - x)
    return quantized_st, loss, idx.reshape(x.shape[0], x.shape[1], x.shape[2])
